# Initial kernel scaffold; baseline (speedup 1.0000x reference)
import jax
import jax.numpy as jnp
from jax.experimental import pallas as pl


def kernel(x, table):
    raise NotImplementedError("write your pallas kernel here")


# speedup vs baseline: 1.1028x; 1.1028x over previous
import functools

import jax
import jax.numpy as jnp
from jax import lax
from jax.experimental import pallas as pl
from jax.experimental.pallas import tpu as pltpu
from jax.experimental.pallas import tpu_sc as plsc

EMBEDDING_DIM = 32
NUM_CORES = 2
NUM_SUBCORES = 16
NUM_WORKERS = NUM_CORES * NUM_SUBCORES
CHUNK = 1600


@functools.partial(jax.jit, static_argnums=(2, 3))
def _gather_rows(table, idx_flat, n_rows, n_chunks):
    mesh = plsc.VectorSubcoreMesh(core_axis_name="c", subcore_axis_name="s")
    bpw = n_rows // NUM_WORKERS

    @functools.partial(
        pl.kernel,
        mesh=mesh,
        out_type=jax.ShapeDtypeStruct((n_rows, EMBEDDING_DIM), jnp.float32),
        scratch_types=[
            pltpu.VMEM((CHUNK,), jnp.int32),
            pltpu.VMEM((CHUNK, EMBEDDING_DIM), jnp.float32),
            pltpu.SemaphoreType.DMA,
        ],
        compiler_params=pltpu.CompilerParams(use_tc_tiling_on_sc=False),
    )
    def k(table_hbm, idx_hbm, out_hbm, idx_v, rows_v, sem):
        wid = lax.axis_index("s") * NUM_CORES + lax.axis_index("c")
        base = wid * bpw

        def body(i, carry):
            off = base + i * CHUNK
            pltpu.sync_copy(idx_hbm.at[pl.ds(off, CHUNK)], idx_v)
            pltpu.async_copy(table_hbm.at[idx_v], rows_v, sem).wait()
            pltpu.sync_copy(rows_v, out_hbm.at[pl.ds(off, CHUNK)])
            return carry

        lax.fori_loop(0, n_chunks, body, 0)

    return k(table, idx_flat)


def kernel(x, table):
    b, h = x.shape
    n_rows = b * h
    idx_flat = x.reshape(n_rows).astype(jnp.int32)
    bpw = n_rows // NUM_WORKERS
    out = _gather_rows(table, idx_flat, n_rows, bpw // CHUNK)
    return out.reshape(b, h, EMBEDDING_DIM)


# speedup vs baseline: 1.1126x; 1.0089x over previous
import functools

import jax
import jax.numpy as jnp
from jax import lax
from jax.experimental import pallas as pl
from jax.experimental.pallas import tpu as pltpu
from jax.experimental.pallas import tpu_sc as plsc

EMBEDDING_DIM = 32
NUM_CORES = 2
NUM_SUBCORES = 16
NUM_WORKERS = NUM_CORES * NUM_SUBCORES
CHUNK = 1600
NBUF = 2


@functools.partial(jax.jit, static_argnums=(2,))
def _gather_rows(table, idx_flat, n_rows):
    mesh = plsc.VectorSubcoreMesh(core_axis_name="c", subcore_axis_name="s")
    bpw = n_rows // NUM_WORKERS
    n_chunks = bpw // CHUNK

    @functools.partial(
        pl.kernel,
        mesh=mesh,
        out_type=jax.ShapeDtypeStruct((n_rows, EMBEDDING_DIM), jnp.float32),
        scratch_types=[
            pltpu.VMEM((bpw,), jnp.int32),
            pltpu.VMEM((CHUNK, EMBEDDING_DIM), jnp.float32),
            pltpu.VMEM((CHUNK, EMBEDDING_DIM), jnp.float32),
            pltpu.SemaphoreType.DMA,
            pltpu.SemaphoreType.DMA,
            pltpu.SemaphoreType.DMA,
            pltpu.SemaphoreType.DMA,
        ],
        compiler_params=pltpu.CompilerParams(use_tc_tiling_on_sc=False),
    )
    def k(table_hbm, idx_hbm, out_hbm, idx_v, rows0, rows1, g0, g1, w0, w1):
        wid = lax.axis_index("s") * NUM_CORES + lax.axis_index("c")
        base = wid * bpw
        rows = (rows0, rows1)
        gsem = (g0, g1)
        wsem = (w0, w1)

        pltpu.sync_copy(idx_hbm.at[pl.ds(base, bpw)], idx_v)

        def gather_desc(i, b):
            return pltpu.make_async_copy(
                table_hbm.at[idx_v.at[pl.ds(i * CHUNK, CHUNK)]], rows[b], gsem[b]
            )

        def write_desc(i, b):
            return pltpu.make_async_copy(
                rows[b], out_hbm.at[pl.ds(base + i * CHUNK, CHUNK)], wsem[b]
            )

        for b in range(NBUF):
            gather_desc(b, b).start()

        def body(g, carry):
            for b in range(NBUF):
                i = g * NBUF + b
                gather_desc(i, b).wait()
                write_desc(i, b).start()
                write_desc(i, b).wait()
                gather_desc(i + NBUF, b).start()
            return carry

        lax.fori_loop(0, n_chunks // NBUF - 1, body, 0)

        for b in range(NBUF):
            i = n_chunks - NBUF + b
            gather_desc(i, b).wait()
            write_desc(i, b).start()
        for b in range(NBUF):
            i = n_chunks - NBUF + b
            write_desc(i, b).wait()

    return k(table, idx_flat)


def kernel(x, table):
    b, h = x.shape
    n_rows = b * h
    idx_flat = x.reshape(n_rows).astype(jnp.int32)
    out = _gather_rows(table, idx_flat, n_rows)
    return out.reshape(b, h, EMBEDDING_DIM)


# speedup vs baseline: 1.8072x; 1.6243x over previous
import functools

import jax
import jax.numpy as jnp
from jax import lax
from jax.experimental import pallas as pl
from jax.experimental.pallas import tpu as pltpu
from jax.experimental.pallas import tpu_sc as plsc

EMBEDDING_DIM = 32
NUM_CORES = 2
NUM_SUBCORES = 16
NUM_WORKERS = NUM_CORES * NUM_SUBCORES
CHUNK_B = 32
NBUF = 2


@functools.partial(jax.jit, static_argnums=(2, 3))
def _gather_rows(table, x, batch, hist):
    mesh = plsc.VectorSubcoreMesh(core_axis_name="c", subcore_axis_name="s")
    bpw = batch // NUM_WORKERS
    n_chunks = bpw // CHUNK_B

    @functools.partial(
        pl.kernel,
        mesh=mesh,
        out_type=jax.ShapeDtypeStruct((batch, hist, EMBEDDING_DIM), jnp.float32),
        scratch_types=[
            pltpu.VMEM((bpw, hist), jnp.int32),
            pltpu.VMEM((CHUNK_B, hist, EMBEDDING_DIM), jnp.float32),
            pltpu.VMEM((CHUNK_B, hist, EMBEDDING_DIM), jnp.float32),
            pltpu.SemaphoreType.DMA,
            pltpu.SemaphoreType.DMA,
            pltpu.SemaphoreType.DMA,
            pltpu.SemaphoreType.DMA,
        ],
        compiler_params=pltpu.CompilerParams(use_tc_tiling_on_sc=False),
    )
    def k(table_hbm, x_hbm, out_hbm, idx_v, rows0, rows1, g0, g1, w0, w1):
        wid = lax.axis_index("s") * NUM_CORES + lax.axis_index("c")
        base = wid * bpw
        rows = (rows0, rows1)
        gsem = (g0, g1)
        wsem = (w0, w1)

        pltpu.sync_copy(x_hbm.at[pl.ds(base, bpw)], idx_v)

        def row_gather_desc(i, j, b):
            return pltpu.make_async_copy(
                table_hbm.at[idx_v.at[i * CHUNK_B + j]],
                rows[b].at[j],
                gsem[b],
            )

        def fire_gathers(i, b):
            def fj(j, c):
                row_gather_desc(i, j, b).start()
                return c

            lax.fori_loop(0, CHUNK_B, fj, 0)

        def drain_gathers(i, b):
            def fj(j, c):
                row_gather_desc(i, j, b).wait()
                return c

            lax.fori_loop(0, CHUNK_B, fj, 0)

        def write_desc(i, b):
            return pltpu.make_async_copy(
                rows[b],
                out_hbm.at[pl.ds(base + i * CHUNK_B, CHUNK_B)],
                wsem[b],
            )

        for b in range(NBUF):
            fire_gathers(b, b)

        def body(g, carry):
            for b in range(NBUF):
                i = g * NBUF + b
                drain_gathers(i, b)
                write_desc(i, b).start()
                write_desc(i, b).wait()
                fire_gathers(i + NBUF, b)
            return carry

        lax.fori_loop(0, n_chunks // NBUF - 1, body, 0)

        for b in range(NBUF):
            i = n_chunks - NBUF + b
            drain_gathers(i, b)
            write_desc(i, b).start()
        for b in range(NBUF):
            i = n_chunks - NBUF + b
            write_desc(i, b).wait()

    return k(table, x)


def kernel(x, table):
    b, h = x.shape
    return _gather_rows(table, x.astype(jnp.int32), b, h)


# speedup vs baseline: 2.2289x; 1.2333x over previous
import functools

import jax
import jax.numpy as jnp
from jax import lax
from jax.experimental import pallas as pl
from jax.experimental.pallas import tpu as pltpu
from jax.experimental.pallas import tpu_sc as plsc

EMBEDDING_DIM = 32
NUM_CORES = 2
NUM_SUBCORES = 16
NUM_WORKERS = NUM_CORES * NUM_SUBCORES
CHUNK_B = 16
NBUF = 2
LANES = 16


@functools.partial(jax.jit, static_argnums=(2, 3))
def _gather_rows(table, x, batch, hist):
    mesh = plsc.VectorSubcoreMesh(core_axis_name="c", subcore_axis_name="s")
    bpw = batch // NUM_WORKERS
    n_chunks = bpw // CHUNK_B
    d8 = EMBEDDING_DIM // 8
    bt = batch // 128

    @functools.partial(
        pl.kernel,
        mesh=mesh,
        out_type=jax.ShapeDtypeStruct((hist, d8, bt, 8, 128), jnp.float32),
        scratch_types=[
            pltpu.VMEM((bpw, hist), jnp.int32),
            pltpu.VMEM((CHUNK_B, hist, EMBEDDING_DIM), jnp.float32),
            pltpu.VMEM((CHUNK_B, hist, EMBEDDING_DIM), jnp.float32),
            pltpu.VMEM((hist, d8, 8, CHUNK_B), jnp.float32),
            pltpu.VMEM((hist, d8, 8, CHUNK_B), jnp.float32),
            pltpu.SemaphoreType.DMA,
            pltpu.SemaphoreType.DMA,
            pltpu.SemaphoreType.DMA,
            pltpu.SemaphoreType.DMA,
        ],
        compiler_params=pltpu.CompilerParams(use_tc_tiling_on_sc=False, needs_layout_passes=False),
    )
    def k(table_hbm, x_hbm, out_hbm, idx_v, rows0, rows1, t0, t1, g0, g1, w0, w1):
        wid = lax.axis_index("s") * NUM_CORES + lax.axis_index("c")
        base = wid * bpw
        rows = (rows0, rows1)
        tbuf = (t0, t1)
        gsem = (g0, g1)
        wsem = (w0, w1)

        pltpu.sync_copy(x_hbm.at[pl.ds(base, bpw)], idx_v)

        iot = lax.iota(jnp.int32, LANES)
        ti_lo = iot // 8
        ti_hi = ti_lo + 2
        dlo = lax.rem(iot, 8)

        def row_gather_desc(i, j, b):
            return pltpu.make_async_copy(
                table_hbm.at[idx_v.at[i * CHUNK_B + j]],
                rows[b].at[j],
                gsem[b],
            )

        def fire_gathers(i, b):
            def fj(j, c):
                row_gather_desc(i, j, b).start()
                return c

            lax.fori_loop(0, CHUNK_B, fj, 0)

        def drain_gathers(i, b):
            def fj(j, c):
                row_gather_desc(i, j, b).wait()
                return c

            lax.fori_loop(0, CHUNK_B, fj, 0)

        HU = 5

        def transpose_chunk(b):
            def fb(j, c):
                jv = jnp.full((LANES,), j, jnp.int32)

                def fh(h5, c2):
                    h0 = h5 * HU
                    for u in range(HU):
                        h = h0 + u
                        lo = rows[b][j, h, pl.ds(0, LANES)]
                        hi = rows[b][j, h, pl.ds(LANES, LANES)]
                        plsc.store_scatter(tbuf[b].at[h], [ti_lo, dlo, jv], lo)
                        plsc.store_scatter(tbuf[b].at[h], [ti_hi, dlo, jv], hi)
                    return c2

                lax.fori_loop(0, hist // HU, fh, 0)
                return c

            lax.fori_loop(0, CHUNK_B, fb, 0)

        def write_desc(i, b):
            gb = base + i * CHUNK_B
            tj = gb // 128
            bl0 = gb % 128
            return pltpu.make_async_copy(
                tbuf[b],
                out_hbm.at[pl.ds(0, hist), pl.ds(0, d8), tj, pl.ds(0, 8),
                           pl.ds(bl0, CHUNK_B)],
                wsem[b],
            )

        for b in range(NBUF):
            fire_gathers(b, b)

        def body(g, carry):
            for b in range(NBUF):
                i = g * NBUF + b
                drain_gathers(i, b)
                transpose_chunk(b)
                write_desc(i, b).start()
                write_desc(i, b).wait()
                fire_gathers(i + NBUF, b)
            return carry

        lax.fori_loop(0, n_chunks // NBUF - 1, body, 0)

        for b in range(NBUF):
            i = n_chunks - NBUF + b
            drain_gathers(i, b)
            transpose_chunk(b)
            write_desc(i, b).start()
        for b in range(NBUF):
            i = n_chunks - NBUF + b
            write_desc(i, b).wait()

    return k(table, x)


def kernel(x, table):
    b, h = x.shape
    out5 = _gather_rows(table, x.astype(jnp.int32), b, h)
    return out5.transpose(2, 4, 0, 1, 3).reshape(b, h, EMBEDDING_DIM)
